# Initial kernel scaffold; baseline (speedup 1.0000x reference)
import jax
import jax.numpy as jnp
from jax.experimental import pallas as pl


def kernel(output, mask, ind, target):
    raise NotImplementedError("write your pallas kernel here")


# speedup vs baseline: 1.5724x; 1.5724x over previous
import jax
import jax.numpy as jnp
from jax import lax
from jax.experimental import pallas as pl
from jax.experimental.pallas import tpu as pltpu
from jax.experimental.pallas import tpu_sc as plsc

B, C, H, W, K = 64, 2, 128, 128, 128
HW = H * W
CHW = C * HW
L = 16
NS = 16
BPW = B // NS


def _sc_body(outflat, maskf, ind, tgt, out,
             indv, maskv, tgtv, idxv0, idxv1, predv0, predv1,
             partv, redv, outv, shared, sem):
    cid = lax.axis_index("c")
    sid = lax.axis_index("s")

    @pl.when(cid == 0)
    def _work():
        acc = jnp.zeros((L,), jnp.float32)
        msum = jnp.zeros((L,), jnp.float32)
        for bl in range(BPW):
            b = sid * BPW + bl
            pltpu.sync_copy(ind.at[b], indv)
            pltpu.sync_copy(maskf.at[b], maskv)
            pltpu.sync_copy(tgt.at[b], tgtv)
            base = b * CHW
            for j in range(8):
                v = indv[pl.ds(L * j, L)] + base
                idxv0[pl.ds(L * j, L)] = v
                idxv1[pl.ds(L * j, L)] = v + HW
            cp0 = pltpu.async_copy(outflat.at[idxv0], predv0, sem)
            cp1 = pltpu.async_copy(outflat.at[idxv1], predv1, sem)
            cp0.wait()
            cp1.wait()
            for j in range(8):
                sl = pl.ds(L * j, L)
                mk = maskv[sl]
                d0 = jnp.abs(predv0[sl] - tgtv[0, sl])
                d1 = jnp.abs(predv1[sl] - tgtv[1, sl])
                acc = acc + (d0 + d1) * mk
                msum = msum + mk
        partv[pl.ds(0, L)] = acc
        partv[pl.ds(L, L)] = msum
        pltpu.sync_copy(partv, shared.at[sid])
        plsc.subcore_barrier()

        @pl.when(sid == 0)
        def _reduce():
            pltpu.sync_copy(shared, redv)
            ta = jnp.zeros((L,), jnp.float32)
            tm = jnp.zeros((L,), jnp.float32)
            for t in range(NS):
                ta = ta + redv[t, pl.ds(0, L)]
                tm = tm + redv[t, pl.ds(L, L)]
            num = jnp.float32(0.0)
            den = jnp.float32(0.0)
            for i in range(L):
                num = num + ta[i]
                den = den + tm[i]
            den = den * jnp.float32(C) + jnp.float32(1e-4)
            numv = jnp.full((L,), num, jnp.float32)
            denv = jnp.full((L,), den, jnp.float32)
            outv[...] = numv / denv
            pltpu.sync_copy(outv, out)


_sc_call = pl.kernel(
    _sc_body,
    out_type=jax.ShapeDtypeStruct((L,), jnp.float32),
    mesh=plsc.VectorSubcoreMesh(core_axis_name="c", subcore_axis_name="s"),
    scratch_types=[
        pltpu.VMEM((K,), jnp.int32),
        pltpu.VMEM((K,), jnp.float32),
        pltpu.VMEM((2, K), jnp.float32),
        pltpu.VMEM((K,), jnp.int32),
        pltpu.VMEM((K,), jnp.int32),
        pltpu.VMEM((K,), jnp.float32),
        pltpu.VMEM((K,), jnp.float32),
        pltpu.VMEM((2 * L,), jnp.float32),
        pltpu.VMEM((NS, 2 * L), jnp.float32),
        pltpu.VMEM((L,), jnp.float32),
        pltpu.VMEM_SHARED((NS, 2 * L), jnp.float32),
        pltpu.SemaphoreType.DMA,
    ],
)


def kernel(output, mask, ind, target):
    outflat = output.reshape(B * C * H * W)
    maskf = mask.astype(jnp.float32)
    ind32 = ind.astype(jnp.int32)
    tgt = jnp.transpose(target, (0, 2, 1))
    res = _sc_call(outflat, maskf, ind32, tgt)
    return res[0]


# speedup vs baseline: 1.9931x; 1.2675x over previous
import jax
import jax.numpy as jnp
from jax import lax
from jax.experimental import pallas as pl
from jax.experimental.pallas import tpu as pltpu
from jax.experimental.pallas import tpu_sc as plsc

B, C, H, W, K = 64, 2, 128, 128, 128
HW = H * W
CHW = C * HW
L = 16
NS = 16
BPW = B // NS


def _sc_body(outflat, maski, ind, tgt, out,
             indall, maskall, tgtall, idxs, preds,
             partv, redv, outv, shared, sem_i, sem_m, sem_t, sem_gs):
    cid = lax.axis_index("c")
    sid = lax.axis_index("s")

    @pl.when(cid == 0)
    def _work():
        b0 = sid * BPW
        st0 = pltpu.async_copy(ind.at[sid], indall, sem_i)
        st1 = pltpu.async_copy(maski.at[sid], maskall, sem_m)
        st2 = pltpu.async_copy(tgt.at[sid], tgtall, sem_t)
        st0.wait()
        gathers = []
        for bl in range(BPW):
            base = (b0 + bl) * CHW
            for j in range(8):
                sl = pl.ds(L * j, L)
                v = indall[pl.ds(K * bl + L * j, L)] + base
                idxs[2 * bl][sl] = v
                idxs[2 * bl + 1][sl] = v + HW
            for r in (2 * bl, 2 * bl + 1):
                gathers.append(
                    pltpu.async_copy(outflat.at[idxs[r]], preds[r], sem_gs[r]))
        st1.wait()
        st2.wait()
        for cp in gathers:
            cp.wait()
        acc = jnp.zeros((L,), jnp.float32)
        msum = jnp.zeros((L,), jnp.float32)
        for bl in range(BPW):
            for j in range(8):
                sl = pl.ds(L * j, L)
                mk = maskall[pl.ds(K * bl + L * j, L)].astype(jnp.float32)
                d0 = jnp.abs(preds[2 * bl][sl]
                             - tgtall[pl.ds(2 * K * bl + L * j, L)])
                d1 = jnp.abs(preds[2 * bl + 1][sl]
                             - tgtall[pl.ds(2 * K * bl + K + L * j, L)])
                acc = acc + (d0 + d1) * mk
                msum = msum + mk
        partv[pl.ds(0, L)] = acc
        partv[pl.ds(L, L)] = msum
        pltpu.sync_copy(partv, shared.at[sid])
        plsc.subcore_barrier()

        @pl.when(sid == 0)
        def _reduce():
            pltpu.sync_copy(shared, redv)
            ta = jnp.zeros((L,), jnp.float32)
            tm = jnp.zeros((L,), jnp.float32)
            for t in range(NS):
                ta = ta + redv[t, pl.ds(0, L)]
                tm = tm + redv[t, pl.ds(L, L)]
            num = jnp.float32(0.0)
            den = jnp.float32(0.0)
            for i in range(L):
                num = num + ta[i]
                den = den + tm[i]
            den = den * jnp.float32(C) + jnp.float32(1e-4)
            numv = jnp.full((L,), num, jnp.float32)
            denv = jnp.full((L,), den, jnp.float32)
            outv[...] = numv / denv
            pltpu.sync_copy(outv, out)


_SCRATCH = [
        pltpu.VMEM((BPW * K,), jnp.int32),
        pltpu.VMEM((BPW * K,), jnp.int32),
        pltpu.VMEM((2 * BPW * K,), jnp.float32),
        [pltpu.VMEM((K,), jnp.int32) for _ in range(2 * BPW)],
        [pltpu.VMEM((K,), jnp.float32) for _ in range(2 * BPW)],
        pltpu.VMEM((2 * L,), jnp.float32),
        pltpu.VMEM((NS, 2 * L), jnp.float32),
        pltpu.VMEM((L,), jnp.float32),
        pltpu.VMEM_SHARED((NS, 2 * L), jnp.float32),
        pltpu.SemaphoreType.DMA,
        pltpu.SemaphoreType.DMA,
        pltpu.SemaphoreType.DMA,
        [pltpu.SemaphoreType.DMA for _ in range(2 * BPW)],
]

_sc_call = pl.kernel(
    _sc_body,
    out_type=jax.ShapeDtypeStruct((L,), jnp.float32),
    mesh=plsc.VectorSubcoreMesh(core_axis_name="c", subcore_axis_name="s"),
    scratch_types=_SCRATCH,
)


def kernel(output, mask, ind, target):
    outflat = output.reshape(B * C * H * W)
    mask32 = mask.astype(jnp.int32).reshape(NS, BPW * K)
    ind32 = ind.astype(jnp.int32).reshape(NS, BPW * K)
    tgt = jnp.transpose(target, (0, 2, 1)).reshape(NS, 2 * BPW * K)
    res = _sc_call(outflat, mask32, ind32, tgt)
    return res[0]
